# baseline (speedup 1.0000x reference)
import jax
import jax.numpy as jnp
from jax.experimental import pallas as pl
from jax.experimental.pallas import tpu as pltpu

H = 400
W = 400
K = 2000
BT = 8
_HI_BITS = 0x3F800000

_SUB_ROWS = 24
_RANK_LO = 189
_RANK_HI = 51


def _count(keys, t):
    return jnp.sum((keys >= t).astype(jnp.int32), axis=(1, 2), keepdims=True)


def _zpad(a, s, axis):
    shape = list(a.shape)
    shape[axis] = s
    return jnp.zeros(shape, a.dtype)


def _shift_up(a, s, axis):
    z = _zpad(a, s, axis)
    if axis == 1:
        return jnp.concatenate([a[:, s:, :], z], axis=1)
    return jnp.concatenate([a[:, :, s:], z], axis=2)


def _shift_down(a, s, axis):
    z = _zpad(a, s, axis)
    if axis == 1:
        return jnp.concatenate([z, a[:, :-s, :]], axis=1)
    return jnp.concatenate([z, a[:, :, :-s]], axis=2)


def _dilate5(a, axis):
    out = a
    for s in (1, 2):
        out = jnp.maximum(out, _shift_up(a, s, axis))
        out = jnp.maximum(out, _shift_down(a, s, axis))
    return out


def _body(pred_ref, mask_ref, out_ref):
    x = pred_ref[...]
    m = mask_ref[...]
    p = m / (1.0 + jnp.exp(-x))
    keys = jax.lax.bitcast_convert_type(p, jnp.int32)

    sub = keys[None, :, :_SUB_ROWS, :]
    rsel = jax.lax.broadcasted_iota(jnp.int32, (2, 1, 1, 1), 0)
    ranks = jnp.where(rsel == 0, _RANK_LO, _RANK_HI)

    def sub_bis(_, carry):
        lo, hi = carry
        mid = jax.lax.shift_right_logical(lo + hi, 1)
        cnt = jnp.sum((sub >= mid).astype(jnp.int32),
                      axis=(2, 3), keepdims=True)
        big = cnt >= ranks
        return (jnp.where(big, mid, lo), jnp.where(big, hi, mid))

    slo, shi = jax.lax.fori_loop(
        0, 16, sub_bis,
        (jnp.zeros((2, BT, 1, 1), jnp.int32),
         jnp.full((2, BT, 1, 1), _HI_BITS, jnp.int32)))
    lo0 = slo[0]
    hi0 = shi[1]

    cl = _count(keys, lo0)
    ch = _count(keys, hi0)
    lo = jnp.where(cl >= K, lo0, 0)
    hi = jnp.where(ch < K, hi0, _HI_BITS)
    lo = jnp.where(ch == K, hi0, jnp.where(cl == K, lo0, lo))
    hi = jnp.where(ch == K, hi0 + 1, jnp.where(cl == K, lo0 + 1, hi))

    def cond(carry):
        lo, hi, it = carry
        return jnp.logical_and(jnp.max(hi - lo) > 1, it < 40)

    def body(carry):
        lo, hi, it = carry
        mid = jax.lax.shift_right_logical(lo + hi, 1)
        cnt = _count(keys, mid)
        big = cnt >= K
        lo2 = jnp.where(big, mid, lo)
        hi2 = jnp.where(cnt == K, mid + 1, jnp.where(big, hi, mid))
        return (lo2, hi2, it + 1)

    thr, _, _ = jax.lax.while_loop(cond, body, (lo, hi, jnp.int32(0)))

    wmax = _dilate5(_dilate5(p, axis=2), axis=1)
    out_ref[...] = wmax >= jax.lax.bitcast_convert_type(thr, jnp.float32)


def _call_chunk(predc, maskc):
    ch = predc.shape[0]
    return pl.pallas_call(
        _body,
        grid=(ch // BT,),
        in_specs=[
            pl.BlockSpec((BT, H, W), lambda i: (i, 0, 0)),
            pl.BlockSpec((BT, H, W), lambda i: (i, 0, 0)),
        ],
        out_specs=pl.BlockSpec((BT, H, W), lambda i: (i, 0, 0)),
        out_shape=jax.ShapeDtypeStruct((ch, H, W), jnp.bool_),
        compiler_params=pltpu.CompilerParams(
            vmem_limit_bytes=100 * 1024 * 1024),
    )(predc, maskc)


def kernel(pred, pred_mask):
    b = pred.shape[0]
    predb = pred.reshape(b, H, W)
    chunk = 32 if b % 32 == 0 else b
    outs = [
        _call_chunk(
            jax.lax.slice_in_dim(predb, i, i + chunk, axis=0),
            jax.lax.slice_in_dim(pred_mask, i, i + chunk, axis=0))
        for i in range(0, b, chunk)
    ]
    return jnp.concatenate(outs, axis=0) if len(outs) > 1 else outs[0]


# speedup vs baseline: 1.4795x; 1.4795x over previous
import jax
import jax.numpy as jnp
from jax.experimental import pallas as pl
from jax.experimental.pallas import tpu as pltpu

H = 400
W = 400
K = 2000
BT = 8
_HI_BITS = 0x3F800000

_SUB_ROWS = 24
_RANK_LO = 189
_RANK_HI = 51


def _count(keys, t):
    return jnp.sum((keys >= t).astype(jnp.int32), axis=(1, 2), keepdims=True)


def _zpad(a, s, axis):
    shape = list(a.shape)
    shape[axis] = s
    return jnp.zeros(shape, a.dtype)


def _shift_up(a, s, axis):
    z = _zpad(a, s, axis)
    if axis == 1:
        return jnp.concatenate([a[:, s:, :], z], axis=1)
    return jnp.concatenate([a[:, :, s:], z], axis=2)


def _shift_down(a, s, axis):
    z = _zpad(a, s, axis)
    if axis == 1:
        return jnp.concatenate([z, a[:, :-s, :]], axis=1)
    return jnp.concatenate([z, a[:, :, :-s]], axis=2)


def _dilate5(a, axis):
    out = a
    for s in (1, 2):
        out = jnp.maximum(out, _shift_up(a, s, axis))
        out = jnp.maximum(out, _shift_down(a, s, axis))
    return out


def _body(p_ref, out_ref):
    p = p_ref[...]
    keys = jax.lax.bitcast_convert_type(p, jnp.int32)

    sub = keys[None, :, :_SUB_ROWS, :]
    rsel = jax.lax.broadcasted_iota(jnp.int32, (2, 1, 1, 1), 0)
    ranks = jnp.where(rsel == 0, _RANK_LO, _RANK_HI)

    def sub_bis(_, carry):
        lo, hi = carry
        mid = jax.lax.shift_right_logical(lo + hi, 1)
        cnt = jnp.sum((sub >= mid).astype(jnp.int32),
                      axis=(2, 3), keepdims=True)
        big = cnt >= ranks
        return (jnp.where(big, mid, lo), jnp.where(big, hi, mid))

    slo, shi = jax.lax.fori_loop(
        0, 16, sub_bis,
        (jnp.zeros((2, BT, 1, 1), jnp.int32),
         jnp.full((2, BT, 1, 1), _HI_BITS, jnp.int32)))
    lo0 = slo[0]
    hi0 = shi[1]

    cl = _count(keys, lo0)
    ch = _count(keys, hi0)
    lo = jnp.where(cl >= K, lo0, 0)
    hi = jnp.where(ch < K, hi0, _HI_BITS)
    lo = jnp.where(ch == K, hi0, jnp.where(cl == K, lo0, lo))
    hi = jnp.where(ch == K, hi0 + 1, jnp.where(cl == K, lo0 + 1, hi))

    def cond(carry):
        lo, hi, it = carry
        return jnp.logical_and(jnp.max(hi - lo) > 1, it < 40)

    def body(carry):
        lo, hi, it = carry
        mid = jax.lax.shift_right_logical(lo + hi, 1)
        cnt = _count(keys, mid)
        big = cnt >= K
        lo2 = jnp.where(big, mid, lo)
        hi2 = jnp.where(cnt == K, mid + 1, jnp.where(big, hi, mid))
        return (lo2, hi2, it + 1)

    thr, _, _ = jax.lax.while_loop(cond, body, (lo, hi, jnp.int32(0)))

    wmax = _dilate5(_dilate5(p, axis=2), axis=1)
    out_ref[...] = wmax >= jax.lax.bitcast_convert_type(thr, jnp.float32)


def kernel(pred, pred_mask):
    b = pred.shape[0]
    p = jax.nn.sigmoid(pred[:, 0]) * pred_mask
    return pl.pallas_call(
        _body,
        grid=(b // BT,),
        in_specs=[
            pl.BlockSpec((BT, H, W), lambda i: (i, 0, 0)),
        ],
        out_specs=pl.BlockSpec((BT, H, W), lambda i: (i, 0, 0)),
        out_shape=jax.ShapeDtypeStruct((b, H, W), jnp.bool_),
        compiler_params=pltpu.CompilerParams(
            vmem_limit_bytes=100 * 1024 * 1024),
    )(p)


# speedup vs baseline: 1.7695x; 1.1960x over previous
import jax
import jax.numpy as jnp
from jax.experimental import pallas as pl

H = 400
W = 400
K = 2000
BT = 8
_HI_BITS = 0x3F800000

_SUB_ROWS = 24
_RANK_LO = 189
_RANK_HI = 51


def _count(keys, t):
    return jnp.sum((keys >= t).astype(jnp.int32), axis=(1, 2), keepdims=True)


def _zpad(a, s, axis):
    shape = list(a.shape)
    shape[axis] = s
    return jnp.zeros(shape, a.dtype)


def _shift_up(a, s, axis):
    z = _zpad(a, s, axis)
    if axis == 1:
        return jnp.concatenate([a[:, s:, :], z], axis=1)
    return jnp.concatenate([a[:, :, s:], z], axis=2)


def _shift_down(a, s, axis):
    z = _zpad(a, s, axis)
    if axis == 1:
        return jnp.concatenate([z, a[:, :-s, :]], axis=1)
    return jnp.concatenate([z, a[:, :, :-s]], axis=2)


def _dilate5(a, axis):
    out = a
    for s in (1, 2):
        out = jnp.maximum(out, _shift_up(a, s, axis))
        out = jnp.maximum(out, _shift_down(a, s, axis))
    return out


def _body(pred_ref, mask_ref, out_ref):
    x = pred_ref[...]
    m = mask_ref[...]
    p = m / (1.0 + jnp.exp(-x))
    keys = jax.lax.bitcast_convert_type(p, jnp.int32)

    sub = keys[None, :, :_SUB_ROWS, :]
    rsel = jax.lax.broadcasted_iota(jnp.int32, (2, 1, 1, 1), 0)
    ranks = jnp.where(rsel == 0, _RANK_LO, _RANK_HI)

    def sub_bis(_, carry):
        lo, hi = carry
        mid = jax.lax.shift_right_logical(lo + hi, 1)
        cnt = jnp.sum((sub >= mid).astype(jnp.int32),
                      axis=(2, 3), keepdims=True)
        big = cnt >= ranks
        return (jnp.where(big, mid, lo), jnp.where(big, hi, mid))

    slo, shi = jax.lax.fori_loop(
        0, 16, sub_bis,
        (jnp.zeros((2, BT, 1, 1), jnp.int32),
         jnp.full((2, BT, 1, 1), _HI_BITS, jnp.int32)))
    lo0 = slo[0]
    hi0 = shi[1]

    cl = _count(keys, lo0)
    ch = _count(keys, hi0)
    lo = jnp.where(cl >= K, lo0, 0)
    hi = jnp.where(ch < K, hi0, _HI_BITS)
    lo = jnp.where(ch == K, hi0, jnp.where(cl == K, lo0, lo))
    hi = jnp.where(ch == K, hi0 + 1, jnp.where(cl == K, lo0 + 1, hi))

    def cond(carry):
        lo, hi, it = carry
        return jnp.logical_and(jnp.max(hi - lo) > 1, it < 40)

    def body(carry):
        lo, hi, it = carry
        mid = jax.lax.shift_right_logical(lo + hi, 1)
        cnt = _count(keys, mid)
        big = cnt >= K
        lo2 = jnp.where(big, mid, lo)
        hi2 = jnp.where(cnt == K, mid + 1, jnp.where(big, hi, mid))
        return (lo2, hi2, it + 1)

    thr, _, _ = jax.lax.while_loop(cond, body, (lo, hi, jnp.int32(0)))

    a = (keys >= thr).astype(jnp.float32)
    hmax = _dilate5(a, axis=2)
    v = _dilate5(hmax, axis=1)
    out_ref[...] = v > 0.0


def kernel(pred, pred_mask):
    b = pred.shape[0]
    predb = pred.reshape(b, H, W)
    return pl.pallas_call(
        _body,
        grid=(b // BT,),
        in_specs=[
            pl.BlockSpec((BT, H, W), lambda i: (i, 0, 0)),
            pl.BlockSpec((BT, H, W), lambda i: (i, 0, 0)),
        ],
        out_specs=pl.BlockSpec((BT, H, W), lambda i: (i, 0, 0)),
        out_shape=jax.ShapeDtypeStruct((b, H, W), jnp.bool_),
    )(predb, pred_mask)


# speedup vs baseline: 1.7903x; 1.0118x over previous
import jax
import jax.numpy as jnp
from jax.experimental import pallas as pl

H = 400
W = 400
K = 2000
BT = 8
_HI_BITS = 0x3F800000

_SUB_ROWS = 24
_RANK_LO = 189
_RANK_HI = 51


def _count(keys, t):
    return jnp.sum((keys >= t).astype(jnp.int32), axis=(1, 2), keepdims=True)


def _zpad(a, s, axis):
    shape = list(a.shape)
    shape[axis] = s
    return jnp.zeros(shape, a.dtype)


def _shift_up(a, s, axis):
    z = _zpad(a, s, axis)
    if axis == 1:
        return jnp.concatenate([a[:, s:, :], z], axis=1)
    return jnp.concatenate([a[:, :, s:], z], axis=2)


def _shift_down(a, s, axis):
    z = _zpad(a, s, axis)
    if axis == 1:
        return jnp.concatenate([z, a[:, :-s, :]], axis=1)
    return jnp.concatenate([z, a[:, :, :-s]], axis=2)


def _dilate5(a, axis):
    out = a
    for s in (1, 2):
        out = jnp.maximum(out, _shift_up(a, s, axis))
        out = jnp.maximum(out, _shift_down(a, s, axis))
    return out


def _body(pred_ref, mask_ref, out_ref):
    x = pred_ref[...]
    m = mask_ref[...]
    p = m / (1.0 + jnp.exp(-x))
    keys = jax.lax.bitcast_convert_type(p, jnp.int32)

    sub = keys[None, :, :_SUB_ROWS, :]
    rsel = jax.lax.broadcasted_iota(jnp.int32, (2, 1, 1, 1), 0)
    ranks = jnp.where(rsel == 0, _RANK_LO, _RANK_HI)

    def sub_bis(_, carry):
        lo, hi = carry
        mid = jax.lax.shift_right_logical(lo + hi, 1)
        cnt = jnp.sum((sub >= mid).astype(jnp.int32),
                      axis=(2, 3), keepdims=True)
        big = cnt >= ranks
        return (jnp.where(big, mid, lo), jnp.where(big, hi, mid))

    slo, shi = jax.lax.fori_loop(
        0, 16, sub_bis,
        (jnp.zeros((2, BT, 1, 1), jnp.int32),
         jnp.full((2, BT, 1, 1), _HI_BITS, jnp.int32)))
    lo0 = slo[0]
    hi0 = shi[1]

    cl = _count(keys, lo0)
    ch = _count(keys, hi0)
    lo = jnp.where(cl >= K, lo0, 0)
    hi = jnp.where(ch < K, hi0, _HI_BITS)
    lo = jnp.where(ch == K, hi0, jnp.where(cl == K, lo0, lo))
    hi = jnp.where(ch == K, hi0 + 1, jnp.where(cl == K, lo0 + 1, hi))

    def cond(carry):
        lo, hi, it = carry
        return jnp.logical_and(jnp.max(hi - lo) > 1, it < 40)

    def body(carry):
        lo, hi, it = carry
        mid = jax.lax.shift_right_logical(lo + hi, 1)
        cnt = _count(keys, mid)
        big = cnt >= K
        lo2 = jnp.where(big, mid, lo)
        hi2 = jnp.where(cnt == K, mid + 1, jnp.where(big, hi, mid))
        return (lo2, hi2, it + 1)

    thr, _, _ = jax.lax.while_loop(cond, body, (lo, hi, jnp.int32(0)))

    a = (keys >= thr).astype(jnp.bfloat16)
    hmax = _dilate5(a, axis=2)
    v = _dilate5(hmax, axis=1)
    out_ref[...] = v > jnp.bfloat16(0.0)


def kernel(pred, pred_mask):
    b = pred.shape[0]
    predb = pred.reshape(b, H, W)
    return pl.pallas_call(
        _body,
        grid=(b // BT,),
        in_specs=[
            pl.BlockSpec((BT, H, W), lambda i: (i, 0, 0)),
            pl.BlockSpec((BT, H, W), lambda i: (i, 0, 0)),
        ],
        out_specs=pl.BlockSpec((BT, H, W), lambda i: (i, 0, 0)),
        out_shape=jax.ShapeDtypeStruct((b, H, W), jnp.bool_),
    )(predb, pred_mask)
